# baseline (speedup 1.0000x reference)
import functools

import jax
import jax.numpy as jnp
from jax import lax
from jax.experimental import pallas as pl
from jax.experimental.pallas import tpu as pltpu
from jax.experimental.pallas import tpu_sc as plsc

EMB = 128
BCH = 16
NB = 8
SG = 16

_GATHER_DNUMS = lax.GatherDimensionNumbers(
    offset_dims=(), collapsed_slice_dims=(0,), start_index_map=(0,)
)


@functools.lru_cache(maxsize=None)
def _build_sc_compute(batch: int):
    info = plsc.get_sparse_core_info()
    nc, ns, nl = info.num_cores, info.num_subcores, info.num_lanes
    nw = nc * ns
    rows_per_w = batch // nw
    n_ch = rows_per_w // BCH
    assert rows_per_w % BCH == 0 and n_ch >= NB and BCH % SG == 0
    mesh = plsc.VectorSubcoreMesh(core_axis_name="c", subcore_axis_name="s")

    @functools.partial(
        pl.kernel,
        mesh=mesh,
        out_type=jax.ShapeDtypeStruct((batch, 4 * EMB), jnp.float32),
        scratch_types=[
            pltpu.VMEM((4, EMB), jnp.float32),
            pltpu.VMEM((6, EMB), jnp.float32),
            pltpu.VMEM((2, EMB), jnp.float32),
            pltpu.VMEM((2, EMB), jnp.float32),
            pltpu.VMEM((rows_per_w,), jnp.int32),
            pltpu.VMEM((NB, BCH, 4 * EMB), jnp.float32),
        ]
        + [pltpu.SemaphoreType.DMA] * (NB + 5),
    )
    def k(st_hbm, ac_hbm, po_hbm, od_hbm, mc_hbm, out_hbm,
          st_v, ac_v, po_v, od_v, mc_v, rows_v, *sems):
        sem_o, sem_i = sems[:NB], sems[NB:]
        wid = lax.axis_index("s") * nc + lax.axis_index("c")
        base = wid * rows_per_w
        loads = [
            pltpu.async_copy(mc_hbm.at[pl.ds(base, rows_per_w)], mc_v, sem_i[0]),
            pltpu.async_copy(st_hbm, st_v, sem_i[1]),
            pltpu.async_copy(ac_hbm, ac_v, sem_i[2]),
            pltpu.async_copy(po_hbm, po_v, sem_i[3]),
            pltpu.async_copy(od_hbm, od_v, sem_i[4]),
        ]
        for h in loads:
            h.wait()
        st1 = [st_v[1, pl.ds(j * nl, nl)] for j in range(8)]
        dP = [po_v[1, pl.ds(j * nl, nl)] - po_v[0, pl.ds(j * nl, nl)]
              for j in range(8)]
        bH = [po_v[0, pl.ds(j * nl, nl)] + od_v[0, pl.ds(j * nl, nl)]
              for j in range(8)]
        bV = [po_v[0, pl.ds(j * nl, nl)] + od_v[1, pl.ds(j * nl, nl)]
              for j in range(8)]
        ac1 = [ac_v[1, pl.ds(j * nl, nl)] for j in range(8)]

        def out_copy(c, b, start):
            src = rows_v.at[b]
            dst = out_hbm.at[pl.ds(pl.multiple_of(base + c * BCH, BCH), BCH)]
            if start:
                return pltpu.async_copy(src, dst, sem_o[b])
            return pltpu.make_async_copy(src, dst, sem_o[b]).wait()

        def supergroup(c, b, sg):
            mc16 = mc_v[pl.ds(c * BCH + sg * SG, SG)]

            def row_body(kk, carry3):
                mc = lax.gather(
                    mc16,
                    jnp.broadcast_to(kk, (nl, 1)).astype(jnp.int32),
                    _GATHER_DNUMS,
                    slice_sizes=(1,),
                    mode=lax.GatherScatterMode.PROMISE_IN_BOUNDS,
                )
                sub = sg * SG + kk
                for seg in range(4):
                    bit = mc >> seg if seg else mc
                    mf = (bit & 1).astype(jnp.float32)
                    for j in range(8):
                        if seg == 0:
                            row = mf * st1[j]
                        elif seg == 1:
                            row = bH[j] + mf * dP[j]
                        elif seg == 2:
                            row = bV[j] + mf * dP[j]
                        else:
                            row = mf * ac1[j]
                        rows_v[b, sub, pl.ds(seg * EMB + j * nl, nl)] = row
                return carry3

            lax.fori_loop(0, SG, row_body, 0)

        def chunk_body(s, carry):
            for b in range(NB):
                c = s * NB + b

                @pl.when(s > 0)
                def _():
                    out_copy(c - NB, b, start=False)

                def gbody(sg, carry2, c=c, b=b):
                    supergroup(c, b, sg)
                    return carry2

                lax.fori_loop(0, BCH // SG, gbody, 0)
                out_copy(c, b, start=True)
            return carry

        lax.fori_loop(0, n_ch // NB, chunk_body, 0)
        for c in range(n_ch - NB, n_ch):
            out_copy(c, c % NB, start=False)

    return k


def kernel(x, street_table, action_table, position_table, order_table):
    batch = x.shape[0]
    weights = jnp.array([1, 2, 4, 8], dtype=jnp.int32)
    mcode = jnp.sum(x.astype(jnp.int32) * weights, axis=1, dtype=jnp.int32)
    return _build_sc_compute(batch)(
        street_table, action_table, position_table, order_table, mcode
    )


# speedup vs baseline: 1.0320x; 1.0320x over previous
import functools

import jax
import jax.numpy as jnp
from jax import lax
from jax.experimental import pallas as pl
from jax.experimental.pallas import tpu as pltpu
from jax.experimental.pallas import tpu_sc as plsc

EMB = 128
BCH = 32
NB = 4
SG = 16

_GATHER_DNUMS = lax.GatherDimensionNumbers(
    offset_dims=(), collapsed_slice_dims=(0,), start_index_map=(0,)
)


@functools.lru_cache(maxsize=None)
def _build_sc_compute(batch: int):
    info = plsc.get_sparse_core_info()
    nc, ns, nl = info.num_cores, info.num_subcores, info.num_lanes
    nw = nc * ns
    rows_per_w = batch // nw
    n_ch = rows_per_w // BCH
    assert rows_per_w % BCH == 0 and n_ch >= NB and BCH % SG == 0
    mesh = plsc.VectorSubcoreMesh(core_axis_name="c", subcore_axis_name="s")

    @functools.partial(
        pl.kernel,
        mesh=mesh,
        out_type=jax.ShapeDtypeStruct((batch, 4 * EMB), jnp.float32),
        scratch_types=[
            pltpu.VMEM((4, EMB), jnp.float32),
            pltpu.VMEM((6, EMB), jnp.float32),
            pltpu.VMEM((2, EMB), jnp.float32),
            pltpu.VMEM((2, EMB), jnp.float32),
            pltpu.VMEM((rows_per_w,), jnp.int32),
            pltpu.VMEM((NB, BCH, 4 * EMB), jnp.float32),
        ]
        + [pltpu.SemaphoreType.DMA] * (NB + 5),
    )
    def k(st_hbm, ac_hbm, po_hbm, od_hbm, mc_hbm, out_hbm,
          st_v, ac_v, po_v, od_v, mc_v, rows_v, *sems):
        sem_o, sem_i = sems[:NB], sems[NB:]
        wid = lax.axis_index("s") * nc + lax.axis_index("c")
        base = wid * rows_per_w
        loads = [
            pltpu.async_copy(mc_hbm.at[pl.ds(base, rows_per_w)], mc_v, sem_i[0]),
            pltpu.async_copy(st_hbm, st_v, sem_i[1]),
            pltpu.async_copy(ac_hbm, ac_v, sem_i[2]),
            pltpu.async_copy(po_hbm, po_v, sem_i[3]),
            pltpu.async_copy(od_hbm, od_v, sem_i[4]),
        ]
        for h in loads:
            h.wait()
        st1 = [st_v[1, pl.ds(j * nl, nl)] for j in range(8)]
        dP = [po_v[1, pl.ds(j * nl, nl)] - po_v[0, pl.ds(j * nl, nl)]
              for j in range(8)]
        bH = [po_v[0, pl.ds(j * nl, nl)] + od_v[0, pl.ds(j * nl, nl)]
              for j in range(8)]
        bV = [po_v[0, pl.ds(j * nl, nl)] + od_v[1, pl.ds(j * nl, nl)]
              for j in range(8)]
        ac1 = [ac_v[1, pl.ds(j * nl, nl)] for j in range(8)]

        def out_copy(c, b, start):
            src = rows_v.at[b]
            dst = out_hbm.at[pl.ds(pl.multiple_of(base + c * BCH, BCH), BCH)]
            if start:
                return pltpu.async_copy(src, dst, sem_o[b])
            return pltpu.make_async_copy(src, dst, sem_o[b]).wait()

        def supergroup(c, b, sg):
            mc16 = mc_v[pl.ds(c * BCH + sg * SG, SG)]

            def row_body(kk, carry3):
                mc = lax.gather(
                    mc16,
                    jnp.broadcast_to(kk, (nl, 1)).astype(jnp.int32),
                    _GATHER_DNUMS,
                    slice_sizes=(1,),
                    mode=lax.GatherScatterMode.PROMISE_IN_BOUNDS,
                )
                sub = sg * SG + kk
                for seg in range(4):
                    bit = mc >> seg if seg else mc
                    mf = (bit & 1).astype(jnp.float32)
                    for j in range(8):
                        if seg == 0:
                            row = mf * st1[j]
                        elif seg == 1:
                            row = bH[j] + mf * dP[j]
                        elif seg == 2:
                            row = bV[j] + mf * dP[j]
                        else:
                            row = mf * ac1[j]
                        rows_v[b, sub, pl.ds(seg * EMB + j * nl, nl)] = row
                return carry3

            lax.fori_loop(0, SG, row_body, 0)

        def chunk_body(s, carry):
            for b in range(NB):
                c = s * NB + b

                @pl.when(s > 0)
                def _():
                    out_copy(c - NB, b, start=False)

                def gbody(sg, carry2, c=c, b=b):
                    supergroup(c, b, sg)
                    return carry2

                lax.fori_loop(0, BCH // SG, gbody, 0)
                out_copy(c, b, start=True)
            return carry

        lax.fori_loop(0, n_ch // NB, chunk_body, 0)
        for c in range(n_ch - NB, n_ch):
            out_copy(c, c % NB, start=False)

    return k


def kernel(x, street_table, action_table, position_table, order_table):
    batch = x.shape[0]
    weights = jnp.array([1, 2, 4, 8], dtype=jnp.int32)
    mcode = jnp.sum(x.astype(jnp.int32) * weights, axis=1, dtype=jnp.int32)
    return _build_sc_compute(batch)(
        street_table, action_table, position_table, order_table, mcode
    )
